# baseline (speedup 1.0000x reference)
import functools

import jax
import jax.numpy as jnp
from jax import lax
from jax.experimental import pallas as pl
from jax.experimental.pallas import tpu as pltpu

_VMEM_LIMIT = 64 * 1024 * 1024


def _conv1_kernel(xf_ref, rhs_ref, y1_ref, sc_ref, st_ref):
    v = xf_ref[0].reshape(30, 64, 128)
    hpe = v[:, 0:32, :].reshape(960, 128)
    hpo = v[:, 32:64, :].reshape(960, 128)
    she = hpe[1:929]
    sho = hpo[1:897]
    lhs = jnp.concatenate(
        [hpe[0:896], she[0:896], hpo[0:896], sho,
         hpe[32:928], she[32:928]], axis=1)
    acc = jnp.dot(lhs, rhs_ref[...], preferred_element_type=jnp.float32)
    ri = lax.broadcasted_iota(jnp.int32, (896, 1), 0)
    valid = (ri % 32) < 28
    y = jnp.where(valid, acc[:, :128], 0.0)
    st_ref[0] = jnp.concatenate(
        [jnp.sum(y, axis=0, keepdims=True),
         jnp.sum(y * y, axis=0, keepdims=True)], axis=0)
    sc_ref[0] = acc[:, 128:].astype(jnp.bfloat16)
    y1_ref[...] = jnp.zeros_like(y1_ref)
    y1_ref[0, 33:929, :] = y.astype(jnp.bfloat16)


def _conv2_kernel(y1_ref, rhs_ref, scale_ref, shift_ref, y2_ref, st_ref):
    yt = y1_ref[0].astype(jnp.float32) * scale_ref[...] + shift_ref[...]
    yt = jnp.maximum(yt, 0.0)
    ri = lax.broadcasted_iota(jnp.int32, (1024, 1), 0)
    h = ri // 32
    w = ri % 32
    interior = (h >= 1) & (h <= 28) & (w >= 1) & (w <= 28)
    xtb = jnp.where(interior, yt, 0.0).astype(jnp.bfloat16)
    v = xtb.reshape(16, 64, 128)
    ve = v[:, 0:32, :].reshape(512, 128)
    vo = v[:, 32:64, :].reshape(512, 128)
    se1, se2 = ve[1:481], ve[2:482]
    so1, so2 = vo[1:481], vo[2:482]
    lhs = jnp.concatenate(
        [ve[0:448], se1[0:448], se2[0:448],
         vo[0:448], so1[0:448], so2[0:448],
         ve[32:480], se1[32:480], se2[32:480],
         vo[32:480], so1[32:480], so2[32:480]], axis=1)
    acc = jnp.dot(lhs, rhs_ref[...], preferred_element_type=jnp.float32)
    rj = lax.broadcasted_iota(jnp.int32, (448, 1), 0)
    ym = jnp.where((rj % 32) < 28, acc, 0.0)
    st_ref[0] = jnp.concatenate(
        [jnp.sum(ym[:, :128], axis=0, keepdims=True)
         + jnp.sum(ym[:, 128:], axis=0, keepdims=True),
         jnp.sum(ym[:, :128] * ym[:, :128], axis=0, keepdims=True)
         + jnp.sum(ym[:, 128:] * ym[:, 128:], axis=0, keepdims=True)],
        axis=0)
    y2_ref[0] = acc.astype(jnp.bfloat16)


def _epilogue_kernel(y2_ref, sc_ref, scale_ref, shift_ref, b3_ref, o_ref):
    s = sc_ref[0].reshape(14, 64, 128)
    sc2 = jnp.concatenate([s[:, 0:32, :].reshape(448, 128),
                           s[:, 32:64, :].reshape(448, 128)], axis=1)
    o_ref[0] = jnp.maximum(
        y2_ref[0].astype(jnp.float32) * scale_ref[...] + shift_ref[...]
        + sc2.astype(jnp.float32) + b3_ref[...], 0.0)


def _bn_fold(stats, count, gamma, beta, eps=1e-5):
    s = jnp.sum(stats[:, 0, :], axis=0)
    sq = jnp.sum(stats[:, 1, :], axis=0)
    mean = s / count
    var = jnp.maximum(sq / count - mean * mean, 0.0)
    scale = gamma * lax.rsqrt(var + eps)
    shift = beta - mean * scale
    return scale.reshape(1, -1), shift.reshape(1, -1)


def kernel(x, w1f, w2p, g1, be1, g2, be2, w3p, b3p):
    N, cin, H, W = x.shape
    Ho, Wo = (H + 2 - 3) // 2 + 1, (W + 2 - 3) // 2 + 1
    M = N * Ho * Wo
    cin_fp = w1f.shape[1]
    cout_p = w1f.shape[2]

    xb = jnp.pad(jnp.transpose(x, (0, 2, 3, 1)).astype(jnp.bfloat16),
                 ((0, 0), (1, 3), (1, 7), (0, 0)))
    xf = xb.reshape(N, 60, 32, 2 * cin)

    def _wt(dy, dx):
        t = (dy // 2) * 2 + (dx // 2)
        slot = (dy % 2) * 2 + (dx % 2)
        return w1f[t, slot * cin:(slot + 1) * cin, :]
    blocks = []
    for dy in range(3):
        for b in range(2):
            top = _wt(dy, 2 * b)
            bot = _wt(dy, 2 * b + 1) if 2 * b + 1 < 3 else jnp.zeros_like(top)
            blocks.append(jnp.concatenate([top, bot], axis=0))
    w1cols = jnp.concatenate(blocks, axis=0)
    sccols = jnp.zeros((6 * 2 * cin, cout_p), jnp.float32)
    sccols = sccols.at[2 * 2 * cin + cin:2 * 2 * cin + 2 * cin].set(w3p[:cin])
    rhs1 = jnp.concatenate([w1cols, sccols], axis=1).astype(jnp.bfloat16)

    y1p, sc, st1 = pl.pallas_call(
        _conv1_kernel,
        out_shape=(jax.ShapeDtypeStruct((N, 1024, cout_p), jnp.bfloat16),
                   jax.ShapeDtypeStruct((N, 896, cout_p), jnp.bfloat16),
                   jax.ShapeDtypeStruct((N, 2, cout_p), jnp.float32)),
        grid=(N,),
        in_specs=[pl.BlockSpec((1, 60, 32, 2 * cin), lambda n: (n, 0, 0, 0)),
                  pl.BlockSpec((6 * 2 * cin, 2 * cout_p), lambda n: (0, 0))],
        out_specs=(pl.BlockSpec((1, 1024, cout_p), lambda n: (n, 0, 0)),
                   pl.BlockSpec((1, 896, cout_p), lambda n: (n, 0, 0)),
                   pl.BlockSpec((1, 2, cout_p), lambda n: (n, 0, 0))),
        compiler_params=pltpu.CompilerParams(
            dimension_semantics=("parallel",),
            vmem_limit_bytes=_VMEM_LIMIT),
    )(xf, rhs1)

    scale1, shift1 = _bn_fold(st1, M, g1, be1)

    z = jnp.zeros((cout_p, cout_p), w2p.dtype)
    pieces = []
    for grp in range(4):
        for b in range(3):
            left = w2p[grp * 3 + b] if grp < 3 else z
            right = w2p[(grp - 1) * 3 + b] if grp >= 1 else z
            pieces.append(jnp.concatenate([left, right], axis=1))
    rhs2 = jnp.concatenate(pieces, axis=0).astype(jnp.bfloat16)

    y2, st2 = pl.pallas_call(
        _conv2_kernel,
        out_shape=(jax.ShapeDtypeStruct((N, 448, 2 * cout_p), jnp.bfloat16),
                   jax.ShapeDtypeStruct((N, 2, cout_p), jnp.float32)),
        grid=(N,),
        in_specs=[pl.BlockSpec((1, 1024, cout_p), lambda n: (n, 0, 0)),
                  pl.BlockSpec((12 * cout_p, 2 * cout_p), lambda n: (0, 0)),
                  pl.BlockSpec((1, cout_p), lambda n: (0, 0)),
                  pl.BlockSpec((1, cout_p), lambda n: (0, 0))],
        out_specs=(pl.BlockSpec((1, 448, 2 * cout_p), lambda n: (n, 0, 0)),
                   pl.BlockSpec((1, 2, cout_p), lambda n: (n, 0, 0))),
        compiler_params=pltpu.CompilerParams(
            dimension_semantics=("parallel",),
            vmem_limit_bytes=_VMEM_LIMIT),
    )(y1p, rhs2, scale1, shift1)

    scale2, shift2 = _bn_fold(st2, M, g2, be2)
    scale2t = jnp.concatenate([scale2, scale2], axis=1)
    shift2t = jnp.concatenate([shift2, shift2], axis=1)
    b3t = jnp.concatenate([b3p, b3p], axis=1)

    chan2 = pl.BlockSpec((1, 2 * cout_p), lambda n: (0, 0))
    out = pl.pallas_call(
        _epilogue_kernel,
        out_shape=jax.ShapeDtypeStruct((N, 448, 2 * cout_p), jnp.float32),
        grid=(N,),
        in_specs=[pl.BlockSpec((1, 448, 2 * cout_p), lambda n: (n, 0, 0)),
                  pl.BlockSpec((1, 896, cout_p), lambda n: (n, 0, 0)),
                  chan2, chan2, chan2],
        out_specs=pl.BlockSpec((1, 448, 2 * cout_p), lambda n: (n, 0, 0)),
        compiler_params=pltpu.CompilerParams(
            dimension_semantics=("parallel",),
            vmem_limit_bytes=_VMEM_LIMIT),
    )(y2, sc, scale2t, shift2t, b3t)

    out = out.reshape(N, 14, 32, 2, cout_p).transpose(0, 4, 1, 3, 2)
    out = out.reshape(N, cout_p, Ho, 32)[:, :, :, :Wo]
    return out


# speedup vs baseline: 1.0748x; 1.0748x over previous
import functools

import jax
import jax.numpy as jnp
from jax import lax
from jax.experimental import pallas as pl
from jax.experimental.pallas import tpu as pltpu

_VMEM_LIMIT = 64 * 1024 * 1024


def _conv1_kernel(xf_ref, rhs_ref, y1_ref, sc_ref, st_ref):
    v = xf_ref[0].reshape(30, 64, 128)
    hpe = v[:, 0:32, :].reshape(960, 128)
    hpo = v[:, 32:64, :].reshape(960, 128)
    she = hpe[1:929]
    sho = hpo[1:897]
    lhs = jnp.concatenate(
        [hpe[0:896], she[0:896], hpo[0:896], sho,
         hpe[32:928], she[32:928]], axis=1)
    acc = jnp.dot(lhs, rhs_ref[...], preferred_element_type=jnp.float32)
    ri = lax.broadcasted_iota(jnp.int32, (896, 1), 0)
    valid = (ri % 32) < 28
    y = jnp.where(valid, acc[:, :128], 0.0)
    st_ref[0] = jnp.concatenate(
        [jnp.sum(y, axis=0, keepdims=True),
         jnp.sum(y * y, axis=0, keepdims=True)], axis=0)
    s = acc[:, 128:].reshape(14, 64, 128)
    sc_ref[0] = jnp.concatenate(
        [s[:, 0:32, :].reshape(448, 128),
         s[:, 32:64, :].reshape(448, 128)], axis=1).astype(jnp.bfloat16)
    y1_ref[...] = jnp.zeros_like(y1_ref)
    y1_ref[0, 33:929, :] = y.astype(jnp.bfloat16)


def _conv2_kernel(y1_ref, rhs_ref, scale_ref, shift_ref, y2_ref, st_ref):
    yt = y1_ref[0].astype(jnp.float32) * scale_ref[...] + shift_ref[...]
    yt = jnp.maximum(yt, 0.0)
    ri = lax.broadcasted_iota(jnp.int32, (1024, 1), 0)
    h = ri // 32
    w = ri % 32
    interior = (h >= 1) & (h <= 28) & (w >= 1) & (w <= 28)
    xtb = jnp.where(interior, yt, 0.0).astype(jnp.bfloat16)
    v = xtb.reshape(16, 64, 128)
    ve = v[:, 0:32, :].reshape(512, 128)
    vo = v[:, 32:64, :].reshape(512, 128)
    se1, se2 = ve[1:481], ve[2:482]
    so1, so2 = vo[1:481], vo[2:482]
    lhs = jnp.concatenate(
        [ve[0:448], se1[0:448], se2[0:448],
         vo[0:448], so1[0:448], so2[0:448],
         ve[32:480], se1[32:480], se2[32:480],
         vo[32:480], so1[32:480], so2[32:480]], axis=1)
    acc = jnp.dot(lhs, rhs_ref[...], preferred_element_type=jnp.float32)
    rj = lax.broadcasted_iota(jnp.int32, (448, 1), 0)
    ym = jnp.where((rj % 32) < 28, acc, 0.0)
    st_ref[0] = jnp.concatenate(
        [jnp.sum(ym[:, :128], axis=0, keepdims=True)
         + jnp.sum(ym[:, 128:], axis=0, keepdims=True),
         jnp.sum(ym[:, :128] * ym[:, :128], axis=0, keepdims=True)
         + jnp.sum(ym[:, 128:] * ym[:, 128:], axis=0, keepdims=True)],
        axis=0)
    y2_ref[0] = acc.astype(jnp.bfloat16)


def _epilogue_kernel(y2_ref, sc_ref, scale_ref, shift_ref, b3_ref, o_ref):
    o_ref[...] = jnp.maximum(
        y2_ref[...].astype(jnp.float32) * scale_ref[...] + shift_ref[...]
        + sc_ref[...].astype(jnp.float32) + b3_ref[...], 0.0)


def _bn_fold(stats, count, gamma, beta, eps=1e-5):
    s = jnp.sum(stats[:, 0, :], axis=0)
    sq = jnp.sum(stats[:, 1, :], axis=0)
    mean = s / count
    var = jnp.maximum(sq / count - mean * mean, 0.0)
    scale = gamma * lax.rsqrt(var + eps)
    shift = beta - mean * scale
    return scale.reshape(1, -1), shift.reshape(1, -1)


def kernel(x, w1f, w2p, g1, be1, g2, be2, w3p, b3p):
    N, cin, H, W = x.shape
    Ho, Wo = (H + 2 - 3) // 2 + 1, (W + 2 - 3) // 2 + 1
    M = N * Ho * Wo
    cin_fp = w1f.shape[1]
    cout_p = w1f.shape[2]

    xb = jnp.pad(jnp.transpose(x, (0, 2, 3, 1)).astype(jnp.bfloat16),
                 ((0, 0), (1, 3), (1, 7), (0, 0)))
    xf = xb.reshape(N, 60, 32, 2 * cin)

    def _wt(dy, dx):
        t = (dy // 2) * 2 + (dx // 2)
        slot = (dy % 2) * 2 + (dx % 2)
        return w1f[t, slot * cin:(slot + 1) * cin, :]
    blocks = []
    for dy in range(3):
        for b in range(2):
            top = _wt(dy, 2 * b)
            bot = _wt(dy, 2 * b + 1) if 2 * b + 1 < 3 else jnp.zeros_like(top)
            blocks.append(jnp.concatenate([top, bot], axis=0))
    w1cols = jnp.concatenate(blocks, axis=0)
    sccols = jnp.zeros((6 * 2 * cin, cout_p), jnp.float32)
    sccols = sccols.at[2 * 2 * cin + cin:2 * 2 * cin + 2 * cin].set(w3p[:cin])
    rhs1 = jnp.concatenate([w1cols, sccols], axis=1).astype(jnp.bfloat16)

    y1p, sc, st1 = pl.pallas_call(
        _conv1_kernel,
        out_shape=(jax.ShapeDtypeStruct((N, 1024, cout_p), jnp.bfloat16),
                   jax.ShapeDtypeStruct((N, 448, 2 * cout_p), jnp.bfloat16),
                   jax.ShapeDtypeStruct((N, 2, cout_p), jnp.float32)),
        grid=(N,),
        in_specs=[pl.BlockSpec((1, 60, 32, 2 * cin), lambda n: (n, 0, 0, 0)),
                  pl.BlockSpec((6 * 2 * cin, 2 * cout_p), lambda n: (0, 0))],
        out_specs=(pl.BlockSpec((1, 1024, cout_p), lambda n: (n, 0, 0)),
                   pl.BlockSpec((1, 448, 2 * cout_p), lambda n: (n, 0, 0)),
                   pl.BlockSpec((1, 2, cout_p), lambda n: (n, 0, 0))),
        compiler_params=pltpu.CompilerParams(
            dimension_semantics=("parallel",),
            vmem_limit_bytes=_VMEM_LIMIT),
    )(xf, rhs1)

    scale1, shift1 = _bn_fold(st1, M, g1, be1)

    z = jnp.zeros((cout_p, cout_p), w2p.dtype)
    pieces = []
    for grp in range(4):
        for b in range(3):
            left = w2p[grp * 3 + b] if grp < 3 else z
            right = w2p[(grp - 1) * 3 + b] if grp >= 1 else z
            pieces.append(jnp.concatenate([left, right], axis=1))
    rhs2 = jnp.concatenate(pieces, axis=0).astype(jnp.bfloat16)

    y2, st2 = pl.pallas_call(
        _conv2_kernel,
        out_shape=(jax.ShapeDtypeStruct((N, 448, 2 * cout_p), jnp.bfloat16),
                   jax.ShapeDtypeStruct((N, 2, cout_p), jnp.float32)),
        grid=(N,),
        in_specs=[pl.BlockSpec((1, 1024, cout_p), lambda n: (n, 0, 0)),
                  pl.BlockSpec((12 * cout_p, 2 * cout_p), lambda n: (0, 0)),
                  pl.BlockSpec((1, cout_p), lambda n: (0, 0)),
                  pl.BlockSpec((1, cout_p), lambda n: (0, 0))],
        out_specs=(pl.BlockSpec((1, 448, 2 * cout_p), lambda n: (n, 0, 0)),
                   pl.BlockSpec((1, 2, cout_p), lambda n: (n, 0, 0))),
        compiler_params=pltpu.CompilerParams(
            dimension_semantics=("parallel",),
            vmem_limit_bytes=_VMEM_LIMIT),
    )(y1p, rhs2, scale1, shift1)

    scale2, shift2 = _bn_fold(st2, M, g2, be2)
    scale2t = jnp.concatenate([scale2, scale2], axis=1)
    shift2t = jnp.concatenate([shift2, shift2], axis=1)
    b3t = jnp.concatenate([b3p, b3p], axis=1)

    Mg = N * 448
    tm = next(t for t in (2048, 448) if Mg % t == 0)
    chan2 = pl.BlockSpec((1, 2 * cout_p), lambda i: (0, 0))
    out = pl.pallas_call(
        _epilogue_kernel,
        out_shape=jax.ShapeDtypeStruct((Mg, 2 * cout_p), jnp.float32),
        grid=(Mg // tm,),
        in_specs=[pl.BlockSpec((tm, 2 * cout_p), lambda i: (i, 0)),
                  pl.BlockSpec((tm, 2 * cout_p), lambda i: (i, 0)),
                  chan2, chan2, chan2],
        out_specs=pl.BlockSpec((tm, 2 * cout_p), lambda i: (i, 0)),
        compiler_params=pltpu.CompilerParams(
            dimension_semantics=("parallel",),
            vmem_limit_bytes=_VMEM_LIMIT),
    )(y2.reshape(Mg, 2 * cout_p), sc.reshape(Mg, 2 * cout_p),
      scale2t, shift2t, b3t)

    out = out.reshape(N, 14, 32, 2, cout_p)[:, :, :Wo, :, :]
    out = out.transpose(0, 4, 1, 3, 2).reshape(N, cout_p, Ho, Wo)
    return out


# speedup vs baseline: 1.2586x; 1.1710x over previous
import functools

import jax
import jax.numpy as jnp
from jax import lax
from jax.experimental import pallas as pl
from jax.experimental.pallas import tpu as pltpu

_VMEM_LIMIT = 64 * 1024 * 1024


def _conv1_kernel(xf_ref, rhs_ref, y1_ref, sc_ref, st_ref):
    v = xf_ref[0].reshape(30, 64, 128)
    hpe = v[:, 0:32, :].reshape(960, 128)
    hpo = v[:, 32:64, :].reshape(960, 128)
    she = hpe[1:929]
    sho = hpo[1:897]
    lhs = jnp.concatenate(
        [hpe[0:896], she[0:896], hpo[0:896], sho,
         hpe[32:928], she[32:928]], axis=1)
    acc = jnp.dot(lhs, rhs_ref[...], preferred_element_type=jnp.float32)
    ri = lax.broadcasted_iota(jnp.int32, (896, 1), 0)
    valid = (ri % 32) < 28
    y = jnp.where(valid, acc[:, :128], 0.0)
    st_ref[0] = jnp.concatenate(
        [jnp.sum(y, axis=0, keepdims=True),
         jnp.sum(y * y, axis=0, keepdims=True)], axis=0)
    sc_ref[0] = acc[:, 128:].astype(jnp.bfloat16)
    y1_ref[...] = jnp.zeros_like(y1_ref)
    y1_ref[0, 33:929, :] = y.astype(jnp.bfloat16)


def _conv2_kernel(y1_ref, rhs_ref, scale_ref, shift_ref, y2_ref, st_ref):
    yt = y1_ref[0].astype(jnp.float32) * scale_ref[...] + shift_ref[...]
    yt = jnp.maximum(yt, 0.0)
    ri = lax.broadcasted_iota(jnp.int32, (1024, 1), 0)
    h = ri // 32
    w = ri % 32
    interior = (h >= 1) & (h <= 28) & (w >= 1) & (w <= 28)
    xtb = jnp.where(interior, yt, 0.0).astype(jnp.bfloat16)
    v = xtb.reshape(16, 64, 128)
    ve = v[:, 0:32, :].reshape(512, 128)
    vo = v[:, 32:64, :].reshape(512, 128)
    se1, se2 = ve[1:481], ve[2:482]
    so1, so2 = vo[1:481], vo[2:482]
    lhs = jnp.concatenate(
        [ve[0:448], se1[0:448], se2[0:448],
         vo[0:448], so1[0:448], so2[0:448],
         ve[32:480], se1[32:480], se2[32:480],
         vo[32:480], so1[32:480], so2[32:480]], axis=1)
    acc = jnp.dot(lhs, rhs_ref[...], preferred_element_type=jnp.float32)
    rj = lax.broadcasted_iota(jnp.int32, (448, 1), 0)
    ym = jnp.where((rj % 32) < 28, acc, 0.0)
    st_ref[0] = jnp.concatenate(
        [jnp.sum(ym[:, :128], axis=0, keepdims=True)
         + jnp.sum(ym[:, 128:], axis=0, keepdims=True),
         jnp.sum(ym[:, :128] * ym[:, :128], axis=0, keepdims=True)
         + jnp.sum(ym[:, 128:] * ym[:, 128:], axis=0, keepdims=True)],
        axis=0)
    ab = acc.astype(jnp.bfloat16)
    y2f = jnp.concatenate([ab[:, :128].reshape(14, 32, 128),
                           ab[:, 128:].reshape(14, 32, 128)], axis=1)
    y2_ref[0] = y2f.reshape(896, 128)


def _epilogue_kernel(y2_ref, sc_ref, scale_ref, shift_ref, b3_ref, o_ref):
    o_ref[...] = jnp.maximum(
        y2_ref[...].astype(jnp.float32) * scale_ref[...] + shift_ref[...]
        + sc_ref[...].astype(jnp.float32) + b3_ref[...], 0.0)


def _bn_fold(stats, count, gamma, beta, eps=1e-5):
    s = jnp.sum(stats[:, 0, :], axis=0)
    sq = jnp.sum(stats[:, 1, :], axis=0)
    mean = s / count
    var = jnp.maximum(sq / count - mean * mean, 0.0)
    scale = gamma * lax.rsqrt(var + eps)
    shift = beta - mean * scale
    return scale.reshape(1, -1), shift.reshape(1, -1)


def kernel(x, w1f, w2p, g1, be1, g2, be2, w3p, b3p):
    N, cin, H, W = x.shape
    Ho, Wo = (H + 2 - 3) // 2 + 1, (W + 2 - 3) // 2 + 1
    M = N * Ho * Wo
    cin_fp = w1f.shape[1]
    cout_p = w1f.shape[2]

    xb = jnp.pad(jnp.transpose(x, (0, 2, 3, 1)).astype(jnp.bfloat16),
                 ((0, 0), (1, 3), (1, 7), (0, 0)))
    xf = xb.reshape(N, 60, 32, 2 * cin)

    def _wt(dy, dx):
        t = (dy // 2) * 2 + (dx // 2)
        slot = (dy % 2) * 2 + (dx % 2)
        return w1f[t, slot * cin:(slot + 1) * cin, :]
    blocks = []
    for dy in range(3):
        for b in range(2):
            top = _wt(dy, 2 * b)
            bot = _wt(dy, 2 * b + 1) if 2 * b + 1 < 3 else jnp.zeros_like(top)
            blocks.append(jnp.concatenate([top, bot], axis=0))
    w1cols = jnp.concatenate(blocks, axis=0)
    sccols = jnp.zeros((6 * 2 * cin, cout_p), jnp.float32)
    sccols = sccols.at[2 * 2 * cin + cin:2 * 2 * cin + 2 * cin].set(w3p[:cin])
    rhs1 = jnp.concatenate([w1cols, sccols], axis=1).astype(jnp.bfloat16)

    y1p, sc, st1 = pl.pallas_call(
        _conv1_kernel,
        out_shape=(jax.ShapeDtypeStruct((N, 1024, cout_p), jnp.bfloat16),
                   jax.ShapeDtypeStruct((N, 896, cout_p), jnp.bfloat16),
                   jax.ShapeDtypeStruct((N, 2, cout_p), jnp.float32)),
        grid=(N,),
        in_specs=[pl.BlockSpec((1, 60, 32, 2 * cin), lambda n: (n, 0, 0, 0)),
                  pl.BlockSpec((6 * 2 * cin, 2 * cout_p), lambda n: (0, 0))],
        out_specs=(pl.BlockSpec((1, 1024, cout_p), lambda n: (n, 0, 0)),
                   pl.BlockSpec((1, 896, cout_p), lambda n: (n, 0, 0)),
                   pl.BlockSpec((1, 2, cout_p), lambda n: (n, 0, 0))),
        compiler_params=pltpu.CompilerParams(
            dimension_semantics=("parallel",),
            vmem_limit_bytes=_VMEM_LIMIT),
    )(xf, rhs1)

    scale1, shift1 = _bn_fold(st1, M, g1, be1)

    z = jnp.zeros((cout_p, cout_p), w2p.dtype)
    pieces = []
    for grp in range(4):
        for b in range(3):
            left = w2p[grp * 3 + b] if grp < 3 else z
            right = w2p[(grp - 1) * 3 + b] if grp >= 1 else z
            pieces.append(jnp.concatenate([left, right], axis=1))
    rhs2 = jnp.concatenate(pieces, axis=0).astype(jnp.bfloat16)

    y2, st2 = pl.pallas_call(
        _conv2_kernel,
        out_shape=(jax.ShapeDtypeStruct((N, 896, cout_p), jnp.bfloat16),
                   jax.ShapeDtypeStruct((N, 2, cout_p), jnp.float32)),
        grid=(N,),
        in_specs=[pl.BlockSpec((1, 1024, cout_p), lambda n: (n, 0, 0)),
                  pl.BlockSpec((12 * cout_p, 2 * cout_p), lambda n: (0, 0)),
                  pl.BlockSpec((1, cout_p), lambda n: (0, 0)),
                  pl.BlockSpec((1, cout_p), lambda n: (0, 0))],
        out_specs=(pl.BlockSpec((1, 896, cout_p), lambda n: (n, 0, 0)),
                   pl.BlockSpec((1, 2, cout_p), lambda n: (n, 0, 0))),
        compiler_params=pltpu.CompilerParams(
            dimension_semantics=("parallel",),
            vmem_limit_bytes=_VMEM_LIMIT),
    )(y1p, rhs2, scale1, shift1)

    scale2, shift2 = _bn_fold(st2, M, g2, be2)

    Mg = N * 896
    tm = next(t for t in (2048, 896) if Mg % t == 0)
    chan = pl.BlockSpec((1, cout_p), lambda i: (0, 0))
    out = pl.pallas_call(
        _epilogue_kernel,
        out_shape=jax.ShapeDtypeStruct((Mg, cout_p), jnp.float32),
        grid=(Mg // tm,),
        in_specs=[pl.BlockSpec((tm, cout_p), lambda i: (i, 0)),
                  pl.BlockSpec((tm, cout_p), lambda i: (i, 0)),
                  chan, chan, chan],
        out_specs=pl.BlockSpec((tm, cout_p), lambda i: (i, 0)),
        compiler_params=pltpu.CompilerParams(
            dimension_semantics=("parallel",),
            vmem_limit_bytes=_VMEM_LIMIT),
    )(y2.reshape(Mg, cout_p), sc.reshape(Mg, cout_p), scale2, shift2, b3p)

    out = out.reshape(N, Ho, 32, cout_p)[:, :, :Wo, :]
    return jnp.transpose(out, (0, 3, 1, 2))


# speedup vs baseline: 1.3432x; 1.0672x over previous
import functools

import jax
import jax.numpy as jnp
from jax import lax
from jax.experimental import pallas as pl
from jax.experimental.pallas import tpu as pltpu

_VMEM_LIMIT = 64 * 1024 * 1024


def _conv1_kernel(xf_ref, rhs_ref, y1_ref, sc_ref, st_ref):
    tp = xf_ref[0]
    tev = tp[:, :, 0:64]
    todd = tp[:, :, 64:128]
    z1 = jnp.zeros((56, 1, 64), jnp.bfloat16)
    z3 = jnp.zeros((56, 3, 64), jnp.bfloat16)
    z4 = jnp.zeros((56, 4, 64), jnp.bfloat16)
    u = jnp.concatenate(
        [jnp.concatenate([z1, todd, z3], axis=1),
         jnp.concatenate([tev, z4], axis=1)], axis=2)
    ur = u.reshape(28, 2, 32, 128)
    zr1 = jnp.zeros((1, 32, 128), jnp.bfloat16)
    hpe = jnp.concatenate([zr1, ur[:, 1, :, :], zr1], axis=0).reshape(960, 128)
    hpo = jnp.concatenate([ur[:, 0, :, :], zr1, zr1], axis=0).reshape(960, 128)
    she = hpe[1:929]
    sho = hpo[1:897]
    lhs = jnp.concatenate(
        [hpe[0:896], she[0:896], hpo[0:896], sho,
         hpe[32:928], she[32:928]], axis=1)
    acc = jnp.dot(lhs, rhs_ref[...], preferred_element_type=jnp.float32)
    ri = lax.broadcasted_iota(jnp.int32, (896, 1), 0)
    valid = (ri % 32) < 28
    y = jnp.where(valid, acc[:, :128], 0.0)
    st_ref[0] = jnp.concatenate(
        [jnp.sum(y, axis=0, keepdims=True),
         jnp.sum(y * y, axis=0, keepdims=True)], axis=0)
    sc_ref[0] = acc[:, 128:].astype(jnp.bfloat16)
    y1_ref[...] = jnp.zeros_like(y1_ref)
    y1_ref[0, 33:929, :] = y.astype(jnp.bfloat16)


def _conv2_kernel(y1_ref, rhs_ref, scale_ref, shift_ref, y2_ref, st_ref):
    yt = y1_ref[0].astype(jnp.float32) * scale_ref[...] + shift_ref[...]
    yt = jnp.maximum(yt, 0.0)
    ri = lax.broadcasted_iota(jnp.int32, (1024, 1), 0)
    h = ri // 32
    w = ri % 32
    interior = (h >= 1) & (h <= 28) & (w >= 1) & (w <= 28)
    xtb = jnp.where(interior, yt, 0.0).astype(jnp.bfloat16)
    v = xtb.reshape(16, 64, 128)
    ve = v[:, 0:32, :].reshape(512, 128)
    vo = v[:, 32:64, :].reshape(512, 128)
    se1, se2 = ve[1:481], ve[2:482]
    so1, so2 = vo[1:481], vo[2:482]
    lhs = jnp.concatenate(
        [ve[0:448], se1[0:448], se2[0:448],
         vo[0:448], so1[0:448], so2[0:448],
         ve[32:480], se1[32:480], se2[32:480],
         vo[32:480], so1[32:480], so2[32:480]], axis=1)
    acc = jnp.dot(lhs, rhs_ref[...], preferred_element_type=jnp.float32)
    rj = lax.broadcasted_iota(jnp.int32, (448, 1), 0)
    ym = jnp.where((rj % 32) < 28, acc, 0.0)
    st_ref[0] = jnp.concatenate(
        [jnp.sum(ym[:, :128], axis=0, keepdims=True)
         + jnp.sum(ym[:, 128:], axis=0, keepdims=True),
         jnp.sum(ym[:, :128] * ym[:, :128], axis=0, keepdims=True)
         + jnp.sum(ym[:, 128:] * ym[:, 128:], axis=0, keepdims=True)],
        axis=0)
    ab = acc.astype(jnp.bfloat16)
    y2f = jnp.concatenate([ab[:, :128].reshape(14, 32, 128),
                           ab[:, 128:].reshape(14, 32, 128)], axis=1)
    y2_ref[0] = y2f.reshape(896, 128)


def _epilogue_kernel(y2_ref, sc_ref, scale_ref, shift_ref, b3_ref, o_ref):
    o_ref[...] = jnp.maximum(
        y2_ref[...].astype(jnp.float32) * scale_ref[...] + shift_ref[...]
        + sc_ref[...].astype(jnp.float32) + b3_ref[...], 0.0)


def _bn_fold(stats, count, gamma, beta, eps=1e-5):
    s = jnp.sum(stats[:, 0, :], axis=0)
    sq = jnp.sum(stats[:, 1, :], axis=0)
    mean = s / count
    var = jnp.maximum(sq / count - mean * mean, 0.0)
    scale = gamma * lax.rsqrt(var + eps)
    shift = beta - mean * scale
    return scale.reshape(1, -1), shift.reshape(1, -1)


def kernel(x, w1f, w2p, g1, be1, g2, be2, w3p, b3p):
    N, cin, H, W = x.shape
    Ho, Wo = (H + 2 - 3) // 2 + 1, (W + 2 - 3) // 2 + 1
    M = N * Ho * Wo
    cin_fp = w1f.shape[1]
    cout_p = w1f.shape[2]

    xq = jnp.transpose(x, (0, 2, 3, 1)).astype(jnp.bfloat16)
    xq = xq.reshape(N, H, W // 2, 2 * cin)

    def _wt(dy, dx):
        t = (dy // 2) * 2 + (dx // 2)
        slot = (dy % 2) * 2 + (dx % 2)
        return w1f[t, slot * cin:(slot + 1) * cin, :]
    blocks = []
    for dy in range(3):
        for b in range(2):
            top = _wt(dy, 2 * b)
            bot = _wt(dy, 2 * b + 1) if 2 * b + 1 < 3 else jnp.zeros_like(top)
            blocks.append(jnp.concatenate([top, bot], axis=0))
    w1cols = jnp.concatenate(blocks, axis=0)
    sccols = jnp.zeros((6 * 2 * cin, cout_p), jnp.float32)
    sccols = sccols.at[2 * 2 * cin + cin:2 * 2 * cin + 2 * cin].set(w3p[:cin])
    rhs1 = jnp.concatenate([w1cols, sccols], axis=1).astype(jnp.bfloat16)

    y1p, sc, st1 = pl.pallas_call(
        _conv1_kernel,
        out_shape=(jax.ShapeDtypeStruct((N, 1024, cout_p), jnp.bfloat16),
                   jax.ShapeDtypeStruct((N, 896, cout_p), jnp.bfloat16),
                   jax.ShapeDtypeStruct((N, 2, cout_p), jnp.float32)),
        grid=(N,),
        in_specs=[pl.BlockSpec((1, H, W // 2, 2 * cin), lambda n: (n, 0, 0, 0)),
                  pl.BlockSpec((6 * 2 * cin, 2 * cout_p), lambda n: (0, 0))],
        out_specs=(pl.BlockSpec((1, 1024, cout_p), lambda n: (n, 0, 0)),
                   pl.BlockSpec((1, 896, cout_p), lambda n: (n, 0, 0)),
                   pl.BlockSpec((1, 2, cout_p), lambda n: (n, 0, 0))),
        compiler_params=pltpu.CompilerParams(
            dimension_semantics=("parallel",),
            vmem_limit_bytes=_VMEM_LIMIT),
    )(xq, rhs1)

    scale1, shift1 = _bn_fold(st1, M, g1, be1)

    z = jnp.zeros((cout_p, cout_p), w2p.dtype)
    pieces = []
    for grp in range(4):
        for b in range(3):
            left = w2p[grp * 3 + b] if grp < 3 else z
            right = w2p[(grp - 1) * 3 + b] if grp >= 1 else z
            pieces.append(jnp.concatenate([left, right], axis=1))
    rhs2 = jnp.concatenate(pieces, axis=0).astype(jnp.bfloat16)

    y2, st2 = pl.pallas_call(
        _conv2_kernel,
        out_shape=(jax.ShapeDtypeStruct((N, 896, cout_p), jnp.bfloat16),
                   jax.ShapeDtypeStruct((N, 2, cout_p), jnp.float32)),
        grid=(N,),
        in_specs=[pl.BlockSpec((1, 1024, cout_p), lambda n: (n, 0, 0)),
                  pl.BlockSpec((12 * cout_p, 2 * cout_p), lambda n: (0, 0)),
                  pl.BlockSpec((1, cout_p), lambda n: (0, 0)),
                  pl.BlockSpec((1, cout_p), lambda n: (0, 0))],
        out_specs=(pl.BlockSpec((1, 896, cout_p), lambda n: (n, 0, 0)),
                   pl.BlockSpec((1, 2, cout_p), lambda n: (n, 0, 0))),
        compiler_params=pltpu.CompilerParams(
            dimension_semantics=("parallel",),
            vmem_limit_bytes=_VMEM_LIMIT),
    )(y1p, rhs2, scale1, shift1)

    scale2, shift2 = _bn_fold(st2, M, g2, be2)

    Mg = N * 896
    tm = next(t for t in (2048, 896) if Mg % t == 0)
    chan = pl.BlockSpec((1, cout_p), lambda i: (0, 0))
    out = pl.pallas_call(
        _epilogue_kernel,
        out_shape=jax.ShapeDtypeStruct((Mg, cout_p), jnp.float32),
        grid=(Mg // tm,),
        in_specs=[pl.BlockSpec((tm, cout_p), lambda i: (i, 0)),
                  pl.BlockSpec((tm, cout_p), lambda i: (i, 0)),
                  chan, chan, chan],
        out_specs=pl.BlockSpec((tm, cout_p), lambda i: (i, 0)),
        compiler_params=pltpu.CompilerParams(
            dimension_semantics=("parallel",),
            vmem_limit_bytes=_VMEM_LIMIT),
    )(y2.reshape(Mg, cout_p), sc.reshape(Mg, cout_p), scale2, shift2, b3p)

    out = out.reshape(N, Ho, 32, cout_p)[:, :, :Wo, :]
    return jnp.transpose(out, (0, 3, 1, 2))


# speedup vs baseline: 1.5519x; 1.1554x over previous
import functools

import jax
import jax.numpy as jnp
from jax import lax
from jax.experimental import pallas as pl
from jax.experimental.pallas import tpu as pltpu

_VMEM_LIMIT = 64 * 1024 * 1024


def _conv1_kernel(xf_ref, rhs_ref, y1_ref, sc_ref, st_ref):
    parts = []
    for im in range(2):
        tp = xf_ref[im]
        tev = tp[:, :, 0:64]
        todd = tp[:, :, 64:128]
        z1 = jnp.zeros((56, 1, 64), jnp.bfloat16)
        z3 = jnp.zeros((56, 3, 64), jnp.bfloat16)
        z4 = jnp.zeros((56, 4, 64), jnp.bfloat16)
        u = jnp.concatenate(
            [jnp.concatenate([z1, todd, z3], axis=1),
             jnp.concatenate([tev, z4], axis=1)], axis=2)
        ur = u.reshape(28, 2, 32, 128)
        zr1 = jnp.zeros((1, 32, 128), jnp.bfloat16)
        hpe = jnp.concatenate([zr1, ur[:, 1, :, :], zr1],
                              axis=0).reshape(960, 128)
        hpo = jnp.concatenate([ur[:, 0, :, :], zr1, zr1],
                              axis=0).reshape(960, 128)
        she = hpe[1:929]
        sho = hpo[1:897]
        parts.append(jnp.concatenate(
            [hpe[0:896], she[0:896], hpo[0:896], sho,
             hpe[32:928], she[32:928]], axis=1))
    lhs = jnp.concatenate(parts, axis=0)
    acc = jnp.dot(lhs, rhs_ref[...], preferred_element_type=jnp.float32)
    ri = lax.broadcasted_iota(jnp.int32, (1792, 1), 0)
    valid = (ri % 32) < 28
    y = jnp.where(valid, acc[:, :128], 0.0)
    st_ref[0] = jnp.concatenate(
        [jnp.sum(y, axis=0, keepdims=True),
         jnp.sum(y * y, axis=0, keepdims=True)], axis=0)
    sc_ref[...] = acc[:, 128:].reshape(2, 896, 128).astype(jnp.bfloat16)
    y1_ref[...] = jnp.zeros_like(y1_ref)
    y1_ref[:, 33:929, :] = y.reshape(2, 896, 128).astype(jnp.bfloat16)


def _conv2_kernel(y1_ref, rhs_ref, scale_ref, shift_ref, y2_ref, st_ref):
    parts = []
    for im in range(2):
        yt = y1_ref[im].astype(jnp.float32) * scale_ref[...] + shift_ref[...]
        yt = jnp.maximum(yt, 0.0)
        ri = lax.broadcasted_iota(jnp.int32, (1024, 1), 0)
        h = ri // 32
        w = ri % 32
        interior = (h >= 1) & (h <= 28) & (w >= 1) & (w <= 28)
        xtb = jnp.where(interior, yt, 0.0).astype(jnp.bfloat16)
        v = xtb.reshape(16, 64, 128)
        ve = v[:, 0:32, :].reshape(512, 128)
        vo = v[:, 32:64, :].reshape(512, 128)
        se1, se2 = ve[1:481], ve[2:482]
        so1, so2 = vo[1:481], vo[2:482]
        parts.append(jnp.concatenate(
            [ve[0:448], se1[0:448], se2[0:448],
             vo[0:448], so1[0:448], so2[0:448],
             ve[32:480], se1[32:480], se2[32:480],
             vo[32:480], so1[32:480], so2[32:480]], axis=1))
    lhs = jnp.concatenate(parts, axis=0)
    acc = jnp.dot(lhs, rhs_ref[...], preferred_element_type=jnp.float32)
    rj = lax.broadcasted_iota(jnp.int32, (896, 1), 0)
    ym = jnp.where((rj % 32) < 28, acc, 0.0)
    st_ref[0] = jnp.concatenate(
        [jnp.sum(ym[:, :128], axis=0, keepdims=True)
         + jnp.sum(ym[:, 128:], axis=0, keepdims=True),
         jnp.sum(ym[:, :128] * ym[:, :128], axis=0, keepdims=True)
         + jnp.sum(ym[:, 128:] * ym[:, 128:], axis=0, keepdims=True)],
        axis=0)
    ab = acc.astype(jnp.bfloat16).reshape(2, 448, 256)
    for im in range(2):
        y2f = jnp.concatenate([ab[im, :, :128].reshape(14, 32, 128),
                               ab[im, :, 128:].reshape(14, 32, 128)], axis=1)
        y2_ref[im] = y2f.reshape(896, 128)


def _epilogue_kernel(y2_ref, sc_ref, scale_ref, shift_ref, b3_ref, o_ref):
    o_ref[...] = jnp.maximum(
        y2_ref[...].astype(jnp.float32) * scale_ref[...] + shift_ref[...]
        + sc_ref[...].astype(jnp.float32) + b3_ref[...], 0.0)


def _bn_fold(stats, count, gamma, beta, eps=1e-5):
    s = jnp.sum(stats[:, 0, :], axis=0)
    sq = jnp.sum(stats[:, 1, :], axis=0)
    mean = s / count
    var = jnp.maximum(sq / count - mean * mean, 0.0)
    scale = gamma * lax.rsqrt(var + eps)
    shift = beta - mean * scale
    return scale.reshape(1, -1), shift.reshape(1, -1)


def kernel(x, w1f, w2p, g1, be1, g2, be2, w3p, b3p):
    N, cin, H, W = x.shape
    Ho, Wo = (H + 2 - 3) // 2 + 1, (W + 2 - 3) // 2 + 1
    M = N * Ho * Wo
    cin_fp = w1f.shape[1]
    cout_p = w1f.shape[2]

    xq = jnp.transpose(x, (0, 2, 3, 1)).astype(jnp.bfloat16)
    xq = xq.reshape(N, H, W // 2, 2 * cin)

    def _wt(dy, dx):
        t = (dy // 2) * 2 + (dx // 2)
        slot = (dy % 2) * 2 + (dx % 2)
        return w1f[t, slot * cin:(slot + 1) * cin, :]
    blocks = []
    for dy in range(3):
        for b in range(2):
            top = _wt(dy, 2 * b)
            bot = _wt(dy, 2 * b + 1) if 2 * b + 1 < 3 else jnp.zeros_like(top)
            blocks.append(jnp.concatenate([top, bot], axis=0))
    w1cols = jnp.concatenate(blocks, axis=0)
    sccols = jnp.zeros((6 * 2 * cin, cout_p), jnp.float32)
    sccols = sccols.at[2 * 2 * cin + cin:2 * 2 * cin + 2 * cin].set(w3p[:cin])
    rhs1 = jnp.concatenate([w1cols, sccols], axis=1).astype(jnp.bfloat16)

    y1p, sc, st1 = pl.pallas_call(
        _conv1_kernel,
        out_shape=(jax.ShapeDtypeStruct((N, 1024, cout_p), jnp.bfloat16),
                   jax.ShapeDtypeStruct((N, 896, cout_p), jnp.bfloat16),
                   jax.ShapeDtypeStruct((N // 2, 2, cout_p), jnp.float32)),
        grid=(N // 2,),
        in_specs=[pl.BlockSpec((2, H, W // 2, 2 * cin), lambda n: (n, 0, 0, 0)),
                  pl.BlockSpec((6 * 2 * cin, 2 * cout_p), lambda n: (0, 0))],
        out_specs=(pl.BlockSpec((2, 1024, cout_p), lambda n: (n, 0, 0)),
                   pl.BlockSpec((2, 896, cout_p), lambda n: (n, 0, 0)),
                   pl.BlockSpec((1, 2, cout_p), lambda n: (n, 0, 0))),
        compiler_params=pltpu.CompilerParams(
            dimension_semantics=("parallel",),
            vmem_limit_bytes=_VMEM_LIMIT),
    )(xq, rhs1)

    scale1, shift1 = _bn_fold(st1, M, g1, be1)

    z = jnp.zeros((cout_p, cout_p), w2p.dtype)
    pieces = []
    for grp in range(4):
        for b in range(3):
            left = w2p[grp * 3 + b] if grp < 3 else z
            right = w2p[(grp - 1) * 3 + b] if grp >= 1 else z
            pieces.append(jnp.concatenate([left, right], axis=1))
    rhs2 = jnp.concatenate(pieces, axis=0).astype(jnp.bfloat16)

    y2, st2 = pl.pallas_call(
        _conv2_kernel,
        out_shape=(jax.ShapeDtypeStruct((N, 896, cout_p), jnp.bfloat16),
                   jax.ShapeDtypeStruct((N // 2, 2, cout_p), jnp.float32)),
        grid=(N // 2,),
        in_specs=[pl.BlockSpec((2, 1024, cout_p), lambda n: (n, 0, 0)),
                  pl.BlockSpec((12 * cout_p, 2 * cout_p), lambda n: (0, 0)),
                  pl.BlockSpec((1, cout_p), lambda n: (0, 0)),
                  pl.BlockSpec((1, cout_p), lambda n: (0, 0))],
        out_specs=(pl.BlockSpec((2, 896, cout_p), lambda n: (n, 0, 0)),
                   pl.BlockSpec((1, 2, cout_p), lambda n: (n, 0, 0))),
        compiler_params=pltpu.CompilerParams(
            dimension_semantics=("parallel",),
            vmem_limit_bytes=_VMEM_LIMIT),
    )(y1p, rhs2, scale1, shift1)

    scale2, shift2 = _bn_fold(st2, M, g2, be2)

    Mg = N * 896
    tm = next(t for t in (2048, 896) if Mg % t == 0)
    chan = pl.BlockSpec((1, cout_p), lambda i: (0, 0))
    out = pl.pallas_call(
        _epilogue_kernel,
        out_shape=jax.ShapeDtypeStruct((Mg, cout_p), jnp.float32),
        grid=(Mg // tm,),
        in_specs=[pl.BlockSpec((tm, cout_p), lambda i: (i, 0)),
                  pl.BlockSpec((tm, cout_p), lambda i: (i, 0)),
                  chan, chan, chan],
        out_specs=pl.BlockSpec((tm, cout_p), lambda i: (i, 0)),
        compiler_params=pltpu.CompilerParams(
            dimension_semantics=("parallel",),
            vmem_limit_bytes=_VMEM_LIMIT),
    )(y2.reshape(Mg, cout_p), sc.reshape(Mg, cout_p), scale2, shift2, b3p)

    out = out.reshape(N, Ho, 32, cout_p)[:, :, :Wo, :]
    return jnp.transpose(out, (0, 3, 1, 2))


# speedup vs baseline: 1.6843x; 1.0853x over previous
import functools

import jax
import jax.numpy as jnp
from jax import lax
from jax.experimental import pallas as pl
from jax.experimental.pallas import tpu as pltpu

_VMEM_LIMIT = 64 * 1024 * 1024


def _conv1_kernel(xf_ref, rhs_ref, y1_ref, sc_ref, st_ref):
    parts = []
    for im in range(4):
        tp = xf_ref[im]
        tev = tp[:, :, 0:64]
        todd = tp[:, :, 64:128]
        z1 = jnp.zeros((56, 1, 64), jnp.bfloat16)
        z3 = jnp.zeros((56, 3, 64), jnp.bfloat16)
        z4 = jnp.zeros((56, 4, 64), jnp.bfloat16)
        u = jnp.concatenate(
            [jnp.concatenate([z1, todd, z3], axis=1),
             jnp.concatenate([tev, z4], axis=1)], axis=2)
        ur = u.reshape(28, 2, 32, 128)
        zr1 = jnp.zeros((1, 32, 128), jnp.bfloat16)
        hpe = jnp.concatenate([zr1, ur[:, 1, :, :], zr1],
                              axis=0).reshape(960, 128)
        hpo = jnp.concatenate([ur[:, 0, :, :], zr1, zr1],
                              axis=0).reshape(960, 128)
        she = hpe[1:929]
        sho = hpo[1:897]
        parts.append(jnp.concatenate(
            [hpe[0:896], she[0:896], hpo[0:896], sho,
             hpe[32:928], she[32:928]], axis=1))
    lhs = jnp.concatenate(parts, axis=0)
    acc = jnp.dot(lhs, rhs_ref[...], preferred_element_type=jnp.float32)
    ri = lax.broadcasted_iota(jnp.int32, (3584, 1), 0)
    valid = (ri % 32) < 28
    y = jnp.where(valid, acc[:, :128], 0.0)
    st_ref[0] = jnp.concatenate(
        [jnp.sum(y, axis=0, keepdims=True),
         jnp.sum(y * y, axis=0, keepdims=True)], axis=0)
    sc_ref[...] = acc[:, 128:].reshape(4, 896, 128).astype(jnp.bfloat16)
    y1_ref[...] = jnp.zeros_like(y1_ref)
    y1_ref[:, 33:929, :] = y.reshape(4, 896, 128).astype(jnp.bfloat16)


def _conv2_kernel(y1_ref, rhs_ref, scale_ref, shift_ref, y2_ref, st_ref):
    parts = []
    for im in range(4):
        yt = y1_ref[im].astype(jnp.float32) * scale_ref[...] + shift_ref[...]
        yt = jnp.maximum(yt, 0.0)
        ri = lax.broadcasted_iota(jnp.int32, (1024, 1), 0)
        h = ri // 32
        w = ri % 32
        interior = (h >= 1) & (h <= 28) & (w >= 1) & (w <= 28)
        xtb = jnp.where(interior, yt, 0.0).astype(jnp.bfloat16)
        v = xtb.reshape(16, 64, 128)
        ve = v[:, 0:32, :].reshape(512, 128)
        vo = v[:, 32:64, :].reshape(512, 128)
        se1, se2 = ve[1:481], ve[2:482]
        so1, so2 = vo[1:481], vo[2:482]
        parts.append(jnp.concatenate(
            [ve[0:448], se1[0:448], se2[0:448],
             vo[0:448], so1[0:448], so2[0:448],
             ve[32:480], se1[32:480], se2[32:480],
             vo[32:480], so1[32:480], so2[32:480]], axis=1))
    lhs = jnp.concatenate(parts, axis=0)
    acc = jnp.dot(lhs, rhs_ref[...], preferred_element_type=jnp.float32)
    rj = lax.broadcasted_iota(jnp.int32, (1792, 1), 0)
    ym = jnp.where((rj % 32) < 28, acc, 0.0)
    st_ref[0] = jnp.concatenate(
        [jnp.sum(ym[:, :128], axis=0, keepdims=True)
         + jnp.sum(ym[:, 128:], axis=0, keepdims=True),
         jnp.sum(ym[:, :128] * ym[:, :128], axis=0, keepdims=True)
         + jnp.sum(ym[:, 128:] * ym[:, 128:], axis=0, keepdims=True)],
        axis=0)
    ab = acc.astype(jnp.bfloat16).reshape(4, 448, 256)
    for im in range(4):
        y2f = jnp.concatenate([ab[im, :, :128].reshape(14, 32, 128),
                               ab[im, :, 128:].reshape(14, 32, 128)], axis=1)
        y2_ref[im] = y2f.reshape(896, 128)


def _epilogue_kernel(y2_ref, sc_ref, scale_ref, shift_ref, b3_ref, o_ref):
    o_ref[...] = jnp.maximum(
        y2_ref[...].astype(jnp.float32) * scale_ref[...] + shift_ref[...]
        + sc_ref[...].astype(jnp.float32) + b3_ref[...], 0.0)


def _bn_fold(stats, count, gamma, beta, eps=1e-5):
    s = jnp.sum(stats[:, 0, :], axis=0)
    sq = jnp.sum(stats[:, 1, :], axis=0)
    mean = s / count
    var = jnp.maximum(sq / count - mean * mean, 0.0)
    scale = gamma * lax.rsqrt(var + eps)
    shift = beta - mean * scale
    return scale.reshape(1, -1), shift.reshape(1, -1)


def kernel(x, w1f, w2p, g1, be1, g2, be2, w3p, b3p):
    N, cin, H, W = x.shape
    Ho, Wo = (H + 2 - 3) // 2 + 1, (W + 2 - 3) // 2 + 1
    M = N * Ho * Wo
    cin_fp = w1f.shape[1]
    cout_p = w1f.shape[2]

    xq = jnp.transpose(x, (0, 2, 3, 1)).astype(jnp.bfloat16)
    xq = xq.reshape(N, H, W // 2, 2 * cin)

    def _wt(dy, dx):
        t = (dy // 2) * 2 + (dx // 2)
        slot = (dy % 2) * 2 + (dx % 2)
        return w1f[t, slot * cin:(slot + 1) * cin, :]
    blocks = []
    for dy in range(3):
        for b in range(2):
            top = _wt(dy, 2 * b)
            bot = _wt(dy, 2 * b + 1) if 2 * b + 1 < 3 else jnp.zeros_like(top)
            blocks.append(jnp.concatenate([top, bot], axis=0))
    w1cols = jnp.concatenate(blocks, axis=0)
    sccols = jnp.zeros((6 * 2 * cin, cout_p), jnp.float32)
    sccols = sccols.at[2 * 2 * cin + cin:2 * 2 * cin + 2 * cin].set(w3p[:cin])
    rhs1 = jnp.concatenate([w1cols, sccols], axis=1).astype(jnp.bfloat16)

    y1p, sc, st1 = pl.pallas_call(
        _conv1_kernel,
        out_shape=(jax.ShapeDtypeStruct((N, 1024, cout_p), jnp.bfloat16),
                   jax.ShapeDtypeStruct((N, 896, cout_p), jnp.bfloat16),
                   jax.ShapeDtypeStruct((N // 4, 2, cout_p), jnp.float32)),
        grid=(N // 4,),
        in_specs=[pl.BlockSpec((4, H, W // 2, 2 * cin), lambda n: (n, 0, 0, 0)),
                  pl.BlockSpec((6 * 2 * cin, 2 * cout_p), lambda n: (0, 0))],
        out_specs=(pl.BlockSpec((4, 1024, cout_p), lambda n: (n, 0, 0)),
                   pl.BlockSpec((4, 896, cout_p), lambda n: (n, 0, 0)),
                   pl.BlockSpec((1, 2, cout_p), lambda n: (n, 0, 0))),
        compiler_params=pltpu.CompilerParams(
            dimension_semantics=("parallel",),
            vmem_limit_bytes=_VMEM_LIMIT),
    )(xq, rhs1)

    scale1, shift1 = _bn_fold(st1, M, g1, be1)

    z = jnp.zeros((cout_p, cout_p), w2p.dtype)
    pieces = []
    for grp in range(4):
        for b in range(3):
            left = w2p[grp * 3 + b] if grp < 3 else z
            right = w2p[(grp - 1) * 3 + b] if grp >= 1 else z
            pieces.append(jnp.concatenate([left, right], axis=1))
    rhs2 = jnp.concatenate(pieces, axis=0).astype(jnp.bfloat16)

    y2, st2 = pl.pallas_call(
        _conv2_kernel,
        out_shape=(jax.ShapeDtypeStruct((N, 896, cout_p), jnp.bfloat16),
                   jax.ShapeDtypeStruct((N // 4, 2, cout_p), jnp.float32)),
        grid=(N // 4,),
        in_specs=[pl.BlockSpec((4, 1024, cout_p), lambda n: (n, 0, 0)),
                  pl.BlockSpec((12 * cout_p, 2 * cout_p), lambda n: (0, 0)),
                  pl.BlockSpec((1, cout_p), lambda n: (0, 0)),
                  pl.BlockSpec((1, cout_p), lambda n: (0, 0))],
        out_specs=(pl.BlockSpec((4, 896, cout_p), lambda n: (n, 0, 0)),
                   pl.BlockSpec((1, 2, cout_p), lambda n: (n, 0, 0))),
        compiler_params=pltpu.CompilerParams(
            dimension_semantics=("parallel",),
            vmem_limit_bytes=_VMEM_LIMIT),
    )(y1p, rhs2, scale1, shift1)

    scale2, shift2 = _bn_fold(st2, M, g2, be2)

    Mg = N * 896
    tm = next(t for t in (2048, 896) if Mg % t == 0)
    chan = pl.BlockSpec((1, cout_p), lambda i: (0, 0))
    out = pl.pallas_call(
        _epilogue_kernel,
        out_shape=jax.ShapeDtypeStruct((Mg, cout_p), jnp.float32),
        grid=(Mg // tm,),
        in_specs=[pl.BlockSpec((tm, cout_p), lambda i: (i, 0)),
                  pl.BlockSpec((tm, cout_p), lambda i: (i, 0)),
                  chan, chan, chan],
        out_specs=pl.BlockSpec((tm, cout_p), lambda i: (i, 0)),
        compiler_params=pltpu.CompilerParams(
            dimension_semantics=("parallel",),
            vmem_limit_bytes=_VMEM_LIMIT),
    )(y2.reshape(Mg, cout_p), sc.reshape(Mg, cout_p), scale2, shift2, b3p)

    out = out.reshape(N, Ho, 32, cout_p)[:, :, :Wo, :]
    return jnp.transpose(out, (0, 3, 1, 2))
